# baseline (speedup 1.0000x reference)
import jax
import jax.numpy as jnp
from jax import lax
from jax.experimental import pallas as pl
from jax.experimental.pallas import tpu as pltpu
from jax.experimental.pallas import tpu_sc as plsc

N = 10000
E = 160000
P = 16
D = 32
MAX_NF = 8
MAX_EF = 8
NF = MAX_EF + 1
T_MP = 3
T_S2S = 6

NC, NS = 2, 16
CHUNK = 125
EROWS = E // CHUNK
RPW = EROWS // (NC * NS)
NROWS = N
RPT = NROWS // NS
NBLK = 2000

_f32 = jnp.float32


def _const_body(ef, w1, b1, w2, b2, src, efq, ewtab, eidx):
    v = jax.nn.relu(jnp.dot(ef[...], w1[...], preferred_element_type=_f32) + b1[...])
    ewtab[...] = jnp.dot(v, w2[...], preferred_element_type=_f32) + b2[...]
    eidx[...] = jnp.clip(efq[...], 0, MAX_EF) * N + src[...]


_const_call = pl.pallas_call(
    _const_body,
    out_shape=[jax.ShapeDtypeStruct((NF, D * D), _f32),
               jax.ShapeDtypeStruct((EROWS, CHUNK), jnp.int32)],
)


def _whole(shape):
    return pl.BlockSpec(shape, lambda i: (0,) * len(shape))


def _project(x, ewt, outp):
    for f in range(NF):
        outp[f] = jnp.dot(x, ewt[f], preferred_element_type=_f32)


def _init_body(pu, pd, nfq, sd, emb, w0, b0, ewt_ref, out0, outp):
    nfi = nfq[...]
    oh = (lax.broadcasted_iota(jnp.int32, (NBLK, NF), 1)
          == jnp.clip(nfi, 0, MAX_NF)).astype(_f32)
    nemb = jnp.dot(oh, emb[...], preferred_element_type=_f32)
    w0v = w0[...]
    x = (jnp.dot(pu[...], w0v[0:P], preferred_element_type=_f32)
         + jnp.dot(pd[...], w0v[P:2 * P], preferred_element_type=_f32)
         + jnp.dot(nemb, w0v[2 * P:2 * P + D], preferred_element_type=_f32)
         + sd[...] * w0v[2 * P + D:2 * P + D + 1]
         + (nfi.astype(_f32) * (1.0 / MAX_NF)) * w0v[2 * P + D + 1:2 * P + D + 2]
         + b0[...])
    o = jax.nn.relu(x)
    out0[...] = o
    _project(o, ewt_ref[...], outp)


_init_call = pl.pallas_call(
    _init_body,
    grid=(N // NBLK,),
    in_specs=[pl.BlockSpec((NBLK, P), lambda i: (i, 0)),
              pl.BlockSpec((NBLK, P), lambda i: (i, 0)),
              pl.BlockSpec((NBLK, 1), lambda i: (i, 0)),
              pl.BlockSpec((NBLK, 1), lambda i: (i, 0)),
              _whole((NF, D)),
              _whole((2 * P + D + 2, D)), _whole((1, D)),
              _whole((NF, D, D))],
    out_specs=[pl.BlockSpec((NBLK, D), lambda i: (i, 0)),
               pl.BlockSpec((NF, NBLK, D), lambda i: (0, i, 0))],
    out_shape=[jax.ShapeDtypeStruct((N, D), _f32),
               jax.ShapeDtypeStruct((NF, N, D), _f32)],
)


NBUF = 4


def _mp_body(outp_hbm, eidx_hbm, dst_hbm, aggs_hbm,
             zbuf, eidx_v, dst_v, rows_v, agg_sh, isem, gsem, ssem):
    cid = lax.axis_index("c")
    sid = lax.axis_index("s")

    base = cid * (EROWS // NC) + sid * RPW
    idx_cp = pltpu.async_copy(eidx_hbm.at[pl.ds(base, RPW)], eidx_v, isem)
    dst_cp = pltpu.async_copy(dst_hbm.at[pl.ds(base, RPW)], dst_v, isem)

    def zr(i, c):
        zbuf[i, pl.ds(0, 16)] = jnp.zeros((16,), _f32)
        zbuf[i, pl.ds(16, 16)] = jnp.zeros((16,), _f32)
        return c

    lax.fori_loop(0, RPT, zr, 0)
    idx_cp.wait()
    dst_cp.wait()
    for k in range(NBUF):
        pltpu.async_copy(outp_hbm.at[eidx_v.at[k]], rows_v.at[k], gsem.at[k])
    pltpu.sync_copy(zbuf, agg_sh.at[pl.ds(sid * RPT, RPT)])
    plsc.subcore_barrier()

    def round_(r, c):
        j0 = r * NBUF
        for k in range(NBUF):
            j = j0 + k
            pltpu.make_async_copy(
                outp_hbm.at[eidx_v.at[j]], rows_v.at[k], gsem.at[k]).wait()
            pltpu.async_copy(rows_v.at[k], agg_sh.at[dst_v.at[j]], ssem.at[k],
                             add=True)
        for k in range(NBUF):
            j = j0 + k

            @pl.when(j + NBUF < RPW)
            def _():
                pltpu.make_async_copy(
                    rows_v.at[k], agg_sh.at[dst_v.at[j]], ssem.at[k]).wait()
                pltpu.async_copy(outp_hbm.at[eidx_v.at[j + NBUF]],
                                 rows_v.at[k], gsem.at[k])
        return c

    lax.fori_loop(0, RPW // NBUF, round_, 0)
    for k in range(NBUF):
        j = RPW - NBUF + k
        pltpu.make_async_copy(
            rows_v.at[k], agg_sh.at[dst_v.at[j]], ssem.at[k]).wait()
    plsc.subcore_barrier()
    pltpu.sync_copy(agg_sh.at[pl.ds(sid * RPT, RPT)], zbuf)
    pltpu.sync_copy(zbuf, aggs_hbm.at[pl.ds(cid * NROWS + sid * RPT, RPT)])


_mp_call = pl.kernel(
    _mp_body,
    out_type=jax.ShapeDtypeStruct((NC * NROWS, D), _f32),
    mesh=plsc.VectorSubcoreMesh(core_axis_name="c", subcore_axis_name="s",
                                num_cores=NC, num_subcores=NS),
    scratch_types=[pltpu.VMEM((RPT, D), _f32),
                   pltpu.VMEM((RPW, CHUNK), jnp.int32),
                   pltpu.VMEM((RPW, CHUNK), jnp.int32),
                   pltpu.VMEM((NBUF, CHUNK, D), _f32),
                   pltpu.VMEM_SHARED((NROWS, D), _f32),
                   pltpu.SemaphoreType.DMA,
                   pltpu.SemaphoreType.DMA((NBUF,)),
                   pltpu.SemaphoreType.DMA((NBUF,))],
    compiler_params=pltpu.CompilerParams(use_tc_tiling_on_sc=False),
)


def _gru_body(a0, a1, h, cb, wih, bih, whh, bhh, ewt_ref, hout, outp):
    hv = h[...]
    m = jax.nn.relu(a0[...] + a1[...] + cb[...])
    gi = jnp.dot(m, wih[...], preferred_element_type=_f32) + bih[...]
    gh = jnp.dot(hv, whh[...], preferred_element_type=_f32) + bhh[...]
    r = jax.nn.sigmoid(gi[:, 0:D] + gh[:, 0:D])
    z = jax.nn.sigmoid(gi[:, D:2 * D] + gh[:, D:2 * D])
    n_ = jnp.tanh(gi[:, 2 * D:3 * D] + r * gh[:, 2 * D:3 * D])
    hnew = (1.0 - z) * n_ + z * hv
    hout[...] = hnew
    _project(hnew, ewt_ref[...], outp)


_gru_call = pl.pallas_call(
    _gru_body,
    grid=(N // NBLK,),
    in_specs=[pl.BlockSpec((NBLK, D), lambda i: (i, 0)),
              pl.BlockSpec((NBLK, D), lambda i: (i + NROWS // NBLK, 0)),
              pl.BlockSpec((NBLK, D), lambda i: (i, 0)),
              _whole((1, D)),
              _whole((D, 3 * D)), _whole((1, 3 * D)),
              _whole((D, 3 * D)), _whole((1, 3 * D)),
              _whole((NF, D, D))],
    out_specs=[pl.BlockSpec((NBLK, D), lambda i: (i, 0)),
               pl.BlockSpec((NF, NBLK, D), lambda i: (0, i, 0))],
    out_shape=[jax.ShapeDtypeStruct((N, D), _f32),
               jax.ShapeDtypeStruct((NF, N, D), _f32)],
)


def _s2s_body(h_ref, wi0, wh0, bi0, bh0, wi1, wh1, bi1, bh1,
              wi2, wh2, bi2, bh2, l1w, l1b, l2w, l2b, out_ref):
    x = h_ref[...]
    wi0v, wh0v, b0v = wi0[...], wh0[...], bi0[...] + bh0[...]
    wi1v, wh1v, b1v = wi1[...], wh1[...], bi1[...] + bh1[...]
    wi2v, wh2v, b2v = wi2[...], wh2[...], bi2[...] + bh2[...]
    l1wv, l2wv = l1w[...], l2w[...]
    zero = jnp.zeros((1, D), _f32)

    def lstm(g, cp):
        i_g = jax.nn.sigmoid(g[:, 0:D])
        f_g = jax.nn.sigmoid(g[:, D:2 * D])
        g_g = jnp.tanh(g[:, 2 * D:3 * D])
        o_g = jax.nn.sigmoid(g[:, 3 * D:4 * D])
        cn = f_g * cp + i_g * g_g
        return o_g * jnp.tanh(cn), cn

    def it(t, carry):
        q, r, h0, c0, h1, c1, h2, c2 = carry
        g = (jnp.dot(q, wi0v[0:D], preferred_element_type=_f32)
             + jnp.dot(r, wi0v[D:2 * D], preferred_element_type=_f32)
             + jnp.dot(h0, wh0v, preferred_element_type=_f32) + b0v)
        h0, c0 = lstm(g, c0)
        g = (jnp.dot(h0, wi1v, preferred_element_type=_f32)
             + jnp.dot(h1, wh1v, preferred_element_type=_f32) + b1v)
        h1, c1 = lstm(g, c1)
        g = (jnp.dot(h1, wi2v, preferred_element_type=_f32)
             + jnp.dot(h2, wh2v, preferred_element_type=_f32) + b2v)
        h2, c2 = lstm(g, c2)
        q = h2
        e = jnp.sum(x * q, axis=1, keepdims=True)
        a = jnp.exp(e - jnp.max(e))
        r = jnp.sum(a * x, axis=0, keepdims=True) / jnp.sum(a)
        return (q, r, h0, c0, h1, c1, h2, c2)

    q, r = lax.fori_loop(0, T_S2S, it, (zero,) * 8)[:2]
    y = jax.nn.relu(jnp.dot(q, l1wv[0:D], preferred_element_type=_f32)
                    + jnp.dot(r, l1wv[D:2 * D], preferred_element_type=_f32)
                    + l1b[...])
    y = jnp.dot(y, l2wv, preferred_element_type=_f32) + l2b[...]
    out_ref[...] = jnp.broadcast_to(y, (8, D))


_s2s_call = pl.pallas_call(
    _s2s_body,
    out_shape=jax.ShapeDtypeStruct((8, D), _f32),
)


def kernel(pos_undirected, pos_directed, params, nfreq, seed, efreq, edge_index):
    p = params
    src = edge_index[0].astype(jnp.int32)
    dst = edge_index[1].astype(jnp.int32)
    efq = efreq.astype(jnp.int32)

    ef_feat = jnp.concatenate(
        [p['edge_freq_emb'], (jnp.arange(NF, dtype=_f32) / MAX_EF)[:, None]], axis=1)
    ewtab, eidx = _const_call(
        ef_feat, p['edge_W1'], p['edge_b1'][None], p['edge_W2'], p['edge_b2'][None],
        src.reshape(EROWS, CHUNK), efq.reshape(EROWS, CHUNK))
    ewt = ewtab.reshape(NF, D, D)
    dstp = dst.reshape(EROWS, CHUNK)

    h, outp = _init_call(
        pos_undirected, pos_directed, nfreq.astype(jnp.int32)[:, None],
        seed.astype(_f32)[:, None], p['node_freq_emb'],
        p['lin0_W'], p['lin0_b'][None], ewt)

    for _ in range(T_MP):
        aggs = _mp_call(outp.reshape(NF * N, D), eidx, dstp)
        h, outp = _gru_call(
            aggs, aggs, h, p['conv_bias'][None],
            p['gru_Wih'], p['gru_bih'][None], p['gru_Whh'], p['gru_bhh'][None],
            ewt)

    y8 = _s2s_call(
        h,
        p['lstm0_Wih'], p['lstm0_Whh'], p['lstm0_bih'][None], p['lstm0_bhh'][None],
        p['lstm1_Wih'], p['lstm1_Whh'], p['lstm1_bih'][None], p['lstm1_bhh'][None],
        p['lstm2_Wih'], p['lstm2_Whh'], p['lstm2_bih'][None], p['lstm2_bhh'][None],
        p['lin1_W'], p['lin1_b'][None], p['lin2_W'], p['lin2_b'][None])
    return y8[0:1]


# speedup vs baseline: 1.2241x; 1.2241x over previous
import jax
import jax.numpy as jnp
from jax import lax
from jax.experimental import pallas as pl
from jax.experimental.pallas import tpu as pltpu
from jax.experimental.pallas import tpu_sc as plsc

N = 10000
E = 160000
P = 16
D = 32
MAX_NF = 8
MAX_EF = 8
NF = MAX_EF + 1
T_MP = 3
T_S2S = 6

NC, NS = 2, 16
CHUNK = 125
EROWS = E // CHUNK
RPW = EROWS // (NC * NS)
NROWS = N
RPT = NROWS // NS
NBLK = 2000

_f32 = jnp.float32


def _const_body(ef, w1, b1, w2, b2, src, efq, ewtab, eidx):
    v = jax.nn.relu(jnp.dot(ef[...], w1[...], preferred_element_type=_f32) + b1[...])
    ewtab[...] = jnp.dot(v, w2[...], preferred_element_type=_f32) + b2[...]
    eidx[...] = src[...] * NF + jnp.clip(efq[...], 0, MAX_EF)


_const_call = pl.pallas_call(
    _const_body,
    out_shape=[jax.ShapeDtypeStruct((NF, D * D), _f32),
               jax.ShapeDtypeStruct((EROWS, CHUNK), jnp.int32)],
)


def _whole(shape):
    return pl.BlockSpec(shape, lambda i: (0,) * len(shape))


def _project(x, wbig, outp):
    outp[...] = jnp.dot(x, wbig, preferred_element_type=_f32)


def _init_body(pu, pd, nfq, sd, emb, w0, b0, wbig, out0, outp):
    nfi = nfq[...]
    oh = (lax.broadcasted_iota(jnp.int32, (NBLK, NF), 1)
          == jnp.clip(nfi, 0, MAX_NF)).astype(_f32)
    nemb = jnp.dot(oh, emb[...], preferred_element_type=_f32)
    w0v = w0[...]
    x = (jnp.dot(pu[...], w0v[0:P], preferred_element_type=_f32)
         + jnp.dot(pd[...], w0v[P:2 * P], preferred_element_type=_f32)
         + jnp.dot(nemb, w0v[2 * P:2 * P + D], preferred_element_type=_f32)
         + sd[...] * w0v[2 * P + D:2 * P + D + 1]
         + (nfi.astype(_f32) * (1.0 / MAX_NF)) * w0v[2 * P + D + 1:2 * P + D + 2]
         + b0[...])
    o = jax.nn.relu(x)
    out0[...] = o
    _project(o, wbig[...], outp)


_init_call = pl.pallas_call(
    _init_body,
    grid=(N // NBLK,),
    in_specs=[pl.BlockSpec((NBLK, P), lambda i: (i, 0)),
              pl.BlockSpec((NBLK, P), lambda i: (i, 0)),
              pl.BlockSpec((NBLK, 1), lambda i: (i, 0)),
              pl.BlockSpec((NBLK, 1), lambda i: (i, 0)),
              _whole((NF, D)),
              _whole((2 * P + D + 2, D)), _whole((1, D)),
              _whole((D, NF * D))],
    out_specs=[pl.BlockSpec((NBLK, D), lambda i: (i, 0)),
               pl.BlockSpec((NBLK, NF * D), lambda i: (i, 0))],
    out_shape=[jax.ShapeDtypeStruct((N, D), _f32),
               jax.ShapeDtypeStruct((N, NF * D), _f32)],
)


NBUF = 4


def _mp_body(outp_hbm, eidx_hbm, dst_hbm, aggs_hbm,
             zbuf, eidx_v, dst_v, rows_v, agg_sh, isem, gsem, ssem):
    cid = lax.axis_index("c")
    sid = lax.axis_index("s")

    base = cid * (EROWS // NC) + sid * RPW
    idx_cp = pltpu.async_copy(eidx_hbm.at[pl.ds(base, RPW)], eidx_v, isem)
    dst_cp = pltpu.async_copy(dst_hbm.at[pl.ds(base, RPW)], dst_v, isem)

    def zr(i, c):
        zbuf[i, pl.ds(0, 16)] = jnp.zeros((16,), _f32)
        zbuf[i, pl.ds(16, 16)] = jnp.zeros((16,), _f32)
        return c

    lax.fori_loop(0, RPT, zr, 0)
    idx_cp.wait()
    dst_cp.wait()
    for k in range(NBUF):
        pltpu.async_copy(outp_hbm.at[eidx_v.at[k]], rows_v.at[k], gsem.at[k])
    pltpu.sync_copy(zbuf, agg_sh.at[pl.ds(sid * RPT, RPT)])
    plsc.subcore_barrier()

    def round_(r, c):
        j0 = r * NBUF
        for k in range(NBUF):
            j = j0 + k
            pltpu.make_async_copy(
                outp_hbm.at[eidx_v.at[j]], rows_v.at[k], gsem.at[k]).wait()
            pltpu.async_copy(rows_v.at[k], agg_sh.at[dst_v.at[j]], ssem.at[k],
                             add=True)
        for k in range(NBUF):
            j = j0 + k

            @pl.when(j + NBUF < RPW)
            def _():
                pltpu.make_async_copy(
                    rows_v.at[k], agg_sh.at[dst_v.at[j]], ssem.at[k]).wait()
                pltpu.async_copy(outp_hbm.at[eidx_v.at[j + NBUF]],
                                 rows_v.at[k], gsem.at[k])
        return c

    lax.fori_loop(0, RPW // NBUF, round_, 0)
    for k in range(NBUF):
        j = RPW - NBUF + k
        pltpu.make_async_copy(
            rows_v.at[k], agg_sh.at[dst_v.at[j]], ssem.at[k]).wait()
    plsc.subcore_barrier()
    pltpu.sync_copy(agg_sh.at[pl.ds(sid * RPT, RPT)], zbuf)
    pltpu.sync_copy(zbuf, aggs_hbm.at[pl.ds(cid * NROWS + sid * RPT, RPT)])


_mp_call = pl.kernel(
    _mp_body,
    out_type=jax.ShapeDtypeStruct((NC * NROWS, D), _f32),
    mesh=plsc.VectorSubcoreMesh(core_axis_name="c", subcore_axis_name="s",
                                num_cores=NC, num_subcores=NS),
    scratch_types=[pltpu.VMEM((RPT, D), _f32),
                   pltpu.VMEM((RPW, CHUNK), jnp.int32),
                   pltpu.VMEM((RPW, CHUNK), jnp.int32),
                   pltpu.VMEM((NBUF, CHUNK, D), _f32),
                   pltpu.VMEM_SHARED((NROWS, D), _f32),
                   pltpu.SemaphoreType.DMA,
                   pltpu.SemaphoreType.DMA((NBUF,)),
                   pltpu.SemaphoreType.DMA((NBUF,))],
    compiler_params=pltpu.CompilerParams(use_tc_tiling_on_sc=False),
)


def _gru_body(a0, a1, h, cb, wih, bih, whh, bhh, wbig, hout, outp):
    hv = h[...]
    m = jax.nn.relu(a0[...] + a1[...] + cb[...])
    gi = jnp.dot(m, wih[...], preferred_element_type=_f32) + bih[...]
    gh = jnp.dot(hv, whh[...], preferred_element_type=_f32) + bhh[...]
    r = jax.nn.sigmoid(gi[:, 0:D] + gh[:, 0:D])
    z = jax.nn.sigmoid(gi[:, D:2 * D] + gh[:, D:2 * D])
    n_ = jnp.tanh(gi[:, 2 * D:3 * D] + r * gh[:, 2 * D:3 * D])
    hnew = (1.0 - z) * n_ + z * hv
    hout[...] = hnew
    _project(hnew, wbig[...], outp)


_gru_call = pl.pallas_call(
    _gru_body,
    grid=(N // NBLK,),
    in_specs=[pl.BlockSpec((NBLK, D), lambda i: (i, 0)),
              pl.BlockSpec((NBLK, D), lambda i: (i + NROWS // NBLK, 0)),
              pl.BlockSpec((NBLK, D), lambda i: (i, 0)),
              _whole((1, D)),
              _whole((D, 3 * D)), _whole((1, 3 * D)),
              _whole((D, 3 * D)), _whole((1, 3 * D)),
              _whole((D, NF * D))],
    out_specs=[pl.BlockSpec((NBLK, D), lambda i: (i, 0)),
               pl.BlockSpec((NBLK, NF * D), lambda i: (i, 0))],
    out_shape=[jax.ShapeDtypeStruct((N, D), _f32),
               jax.ShapeDtypeStruct((N, NF * D), _f32)],
)


def _s2s_body(h_ref, wi0, wh0, bi0, bh0, wi1, wh1, bi1, bh1,
              wi2, wh2, bi2, bh2, l1w, l1b, l2w, l2b, out_ref):
    x = h_ref[...]
    wi0v, wh0v, b0v = wi0[...], wh0[...], bi0[...] + bh0[...]
    wi1v, wh1v, b1v = wi1[...], wh1[...], bi1[...] + bh1[...]
    wi2v, wh2v, b2v = wi2[...], wh2[...], bi2[...] + bh2[...]
    l1wv, l2wv = l1w[...], l2w[...]
    zero = jnp.zeros((1, D), _f32)

    def lstm(g, cp):
        i_g = jax.nn.sigmoid(g[:, 0:D])
        f_g = jax.nn.sigmoid(g[:, D:2 * D])
        g_g = jnp.tanh(g[:, 2 * D:3 * D])
        o_g = jax.nn.sigmoid(g[:, 3 * D:4 * D])
        cn = f_g * cp + i_g * g_g
        return o_g * jnp.tanh(cn), cn

    def it(t, carry):
        q, r, h0, c0, h1, c1, h2, c2 = carry
        g = (jnp.dot(q, wi0v[0:D], preferred_element_type=_f32)
             + jnp.dot(r, wi0v[D:2 * D], preferred_element_type=_f32)
             + jnp.dot(h0, wh0v, preferred_element_type=_f32) + b0v)
        h0, c0 = lstm(g, c0)
        g = (jnp.dot(h0, wi1v, preferred_element_type=_f32)
             + jnp.dot(h1, wh1v, preferred_element_type=_f32) + b1v)
        h1, c1 = lstm(g, c1)
        g = (jnp.dot(h1, wi2v, preferred_element_type=_f32)
             + jnp.dot(h2, wh2v, preferred_element_type=_f32) + b2v)
        h2, c2 = lstm(g, c2)
        q = h2
        e = jnp.sum(x * q, axis=1, keepdims=True)
        a = jnp.exp(e - jnp.max(e))
        r = jnp.sum(a * x, axis=0, keepdims=True) / jnp.sum(a)
        return (q, r, h0, c0, h1, c1, h2, c2)

    q, r = lax.fori_loop(0, T_S2S, it, (zero,) * 8)[:2]
    y = jax.nn.relu(jnp.dot(q, l1wv[0:D], preferred_element_type=_f32)
                    + jnp.dot(r, l1wv[D:2 * D], preferred_element_type=_f32)
                    + l1b[...])
    y = jnp.dot(y, l2wv, preferred_element_type=_f32) + l2b[...]
    out_ref[...] = jnp.broadcast_to(y, (8, D))


_s2s_call = pl.pallas_call(
    _s2s_body,
    out_shape=jax.ShapeDtypeStruct((8, D), _f32),
)


def kernel(pos_undirected, pos_directed, params, nfreq, seed, efreq, edge_index):
    p = params
    src = edge_index[0].astype(jnp.int32)
    dst = edge_index[1].astype(jnp.int32)
    efq = efreq.astype(jnp.int32)

    ef_feat = jnp.concatenate(
        [p['edge_freq_emb'], (jnp.arange(NF, dtype=_f32) / MAX_EF)[:, None]], axis=1)
    ewtab, eidx = _const_call(
        ef_feat, p['edge_W1'], p['edge_b1'][None], p['edge_W2'], p['edge_b2'][None],
        src.reshape(EROWS, CHUNK), efq.reshape(EROWS, CHUNK))
    wbig = ewtab.reshape(NF, D, D).transpose(1, 0, 2).reshape(D, NF * D)
    dstp = dst.reshape(EROWS, CHUNK)

    h, outp = _init_call(
        pos_undirected, pos_directed, nfreq.astype(jnp.int32)[:, None],
        seed.astype(_f32)[:, None], p['node_freq_emb'],
        p['lin0_W'], p['lin0_b'][None], wbig)

    for _ in range(T_MP):
        aggs = _mp_call(outp.reshape(N * NF, D), eidx, dstp)
        h, outp = _gru_call(
            aggs, aggs, h, p['conv_bias'][None],
            p['gru_Wih'], p['gru_bih'][None], p['gru_Whh'], p['gru_bhh'][None],
            wbig)

    y8 = _s2s_call(
        h,
        p['lstm0_Wih'], p['lstm0_Whh'], p['lstm0_bih'][None], p['lstm0_bhh'][None],
        p['lstm1_Wih'], p['lstm1_Whh'], p['lstm1_bih'][None], p['lstm1_bhh'][None],
        p['lstm2_Wih'], p['lstm2_Whh'], p['lstm2_bih'][None], p['lstm2_bhh'][None],
        p['lin1_W'], p['lin1_b'][None], p['lin2_W'], p['lin2_b'][None])
    return y8[0:1]


# speedup vs baseline: 1.3063x; 1.0671x over previous
import jax
import jax.numpy as jnp
from jax import lax
from jax.experimental import pallas as pl
from jax.experimental.pallas import tpu as pltpu
from jax.experimental.pallas import tpu_sc as plsc

N = 10000
E = 160000
P = 16
D = 32
MAX_NF = 8
MAX_EF = 8
NF = MAX_EF + 1
T_MP = 3
T_S2S = 6

NC, NS = 2, 16
CHUNK = 125
EROWS = E // CHUNK
RPW = EROWS // (NC * NS)
NROWS = N
RPT = NROWS // NS
NBLK = 2000

_f32 = jnp.float32


def _const_body(ef, w1, b1, w2, b2, src, efq, ewtab, eidx):
    v = jax.nn.relu(jnp.dot(ef[...], w1[...], preferred_element_type=_f32) + b1[...])
    ewtab[...] = jnp.dot(v, w2[...], preferred_element_type=_f32) + b2[...]
    eidx[...] = src[...] * NF + jnp.clip(efq[...], 0, MAX_EF)


_const_call = pl.pallas_call(
    _const_body,
    out_shape=[jax.ShapeDtypeStruct((NF, D * D), _f32),
               jax.ShapeDtypeStruct((EROWS, CHUNK), jnp.int32)],
)


def _whole(shape):
    return pl.BlockSpec(shape, lambda i: (0,) * len(shape))


def _project(x, wbig, outp):
    outp[...] = jnp.dot(x, wbig, preferred_element_type=_f32)


FDIM = 2 * P + 2


def _init_body(feats, embw, wcat, b0, wbig, out0, outp):
    fv = feats[...]
    nfi = jnp.clip(fv[:, FDIM - 1:FDIM].astype(jnp.int32), 0, MAX_NF)
    oh = (lax.broadcasted_iota(jnp.int32, (NBLK, NF), 1) == nfi).astype(_f32)
    x = (jnp.dot(fv, wcat[...], preferred_element_type=_f32)
         + jnp.dot(oh, embw[...], preferred_element_type=_f32)
         + b0[...])
    o = jax.nn.relu(x)
    out0[...] = o
    _project(o, wbig[...], outp)


_init_call = pl.pallas_call(
    _init_body,
    grid=(N // NBLK,),
    in_specs=[pl.BlockSpec((NBLK, FDIM), lambda i: (i, 0)),
              _whole((NF, D)),
              _whole((FDIM, D)), _whole((1, D)),
              _whole((D, NF * D))],
    out_specs=[pl.BlockSpec((NBLK, D), lambda i: (i, 0)),
               pl.BlockSpec((NBLK, NF * D), lambda i: (i, 0))],
    out_shape=[jax.ShapeDtypeStruct((N, D), _f32),
               jax.ShapeDtypeStruct((N, NF * D), _f32)],
)


NBUF = 4


def _mp_body(outp_hbm, eidx_hbm, dst_hbm, aggs_hbm,
             zbuf, eidx_v, dst_v, rows_v, agg_sh, isem, gsem, ssem):
    cid = lax.axis_index("c")
    sid = lax.axis_index("s")

    base = cid * (EROWS // NC) + sid * RPW
    idx_cp = pltpu.async_copy(eidx_hbm.at[pl.ds(base, RPW)], eidx_v, isem)
    dst_cp = pltpu.async_copy(dst_hbm.at[pl.ds(base, RPW)], dst_v, isem)

    def zr(i, c):
        zbuf[i, pl.ds(0, 16)] = jnp.zeros((16,), _f32)
        zbuf[i, pl.ds(16, 16)] = jnp.zeros((16,), _f32)
        return c

    lax.fori_loop(0, RPT, zr, 0)
    idx_cp.wait()
    dst_cp.wait()
    for k in range(NBUF):
        pltpu.async_copy(outp_hbm.at[eidx_v.at[k]], rows_v.at[k], gsem.at[k])
    pltpu.sync_copy(zbuf, agg_sh.at[pl.ds(sid * RPT, RPT)])
    plsc.subcore_barrier()

    def round_(r, c):
        j0 = r * NBUF
        for k in range(NBUF):
            j = j0 + k
            pltpu.make_async_copy(
                outp_hbm.at[eidx_v.at[j]], rows_v.at[k], gsem.at[k]).wait()
            pltpu.async_copy(rows_v.at[k], agg_sh.at[dst_v.at[j]], ssem.at[k],
                             add=True)
        for k in range(NBUF):
            j = j0 + k

            @pl.when(j + NBUF < RPW)
            def _():
                pltpu.make_async_copy(
                    rows_v.at[k], agg_sh.at[dst_v.at[j]], ssem.at[k]).wait()
                pltpu.async_copy(outp_hbm.at[eidx_v.at[j + NBUF]],
                                 rows_v.at[k], gsem.at[k])
        return c

    lax.fori_loop(0, RPW // NBUF, round_, 0)
    for k in range(NBUF):
        j = RPW - NBUF + k
        pltpu.make_async_copy(
            rows_v.at[k], agg_sh.at[dst_v.at[j]], ssem.at[k]).wait()
    plsc.subcore_barrier()
    pltpu.sync_copy(agg_sh.at[pl.ds(sid * RPT, RPT)], zbuf)
    pltpu.sync_copy(zbuf, aggs_hbm.at[pl.ds(cid * NROWS + sid * RPT, RPT)])


_mp_call = pl.kernel(
    _mp_body,
    out_type=jax.ShapeDtypeStruct((NC * NROWS, D), _f32),
    mesh=plsc.VectorSubcoreMesh(core_axis_name="c", subcore_axis_name="s",
                                num_cores=NC, num_subcores=NS),
    scratch_types=[pltpu.VMEM((RPT, D), _f32),
                   pltpu.VMEM((RPW, CHUNK), jnp.int32),
                   pltpu.VMEM((RPW, CHUNK), jnp.int32),
                   pltpu.VMEM((NBUF, CHUNK, D), _f32),
                   pltpu.VMEM_SHARED((NROWS, D), _f32),
                   pltpu.SemaphoreType.DMA,
                   pltpu.SemaphoreType.DMA((NBUF,)),
                   pltpu.SemaphoreType.DMA((NBUF,))],
    compiler_params=pltpu.CompilerParams(use_tc_tiling_on_sc=False),
)


def _gru_body(a0, a1, h, cb, wih, bih, whh, bhh, wbig, hout, outp):
    hv = h[...]
    m = jax.nn.relu(a0[...] + a1[...] + cb[...])
    gi = jnp.dot(m, wih[...], preferred_element_type=_f32) + bih[...]
    gh = jnp.dot(hv, whh[...], preferred_element_type=_f32) + bhh[...]
    r = jax.nn.sigmoid(gi[:, 0:D] + gh[:, 0:D])
    z = jax.nn.sigmoid(gi[:, D:2 * D] + gh[:, D:2 * D])
    n_ = jnp.tanh(gi[:, 2 * D:3 * D] + r * gh[:, 2 * D:3 * D])
    hnew = (1.0 - z) * n_ + z * hv
    hout[...] = hnew
    _project(hnew, wbig[...], outp)


_gru_call = pl.pallas_call(
    _gru_body,
    grid=(N // NBLK,),
    in_specs=[pl.BlockSpec((NBLK, D), lambda i: (i, 0)),
              pl.BlockSpec((NBLK, D), lambda i: (i + NROWS // NBLK, 0)),
              pl.BlockSpec((NBLK, D), lambda i: (i, 0)),
              _whole((1, D)),
              _whole((D, 3 * D)), _whole((1, 3 * D)),
              _whole((D, 3 * D)), _whole((1, 3 * D)),
              _whole((D, NF * D))],
    out_specs=[pl.BlockSpec((NBLK, D), lambda i: (i, 0)),
               pl.BlockSpec((NBLK, NF * D), lambda i: (i, 0))],
    out_shape=[jax.ShapeDtypeStruct((N, D), _f32),
               jax.ShapeDtypeStruct((N, NF * D), _f32)],
)


def _s2s_body(h_ref, wi0, wh0, bi0, bh0, wi1, wh1, bi1, bh1,
              wi2, wh2, bi2, bh2, l1w, l1b, l2w, l2b, out_ref):
    x = h_ref[...]
    wi0v, wh0v, b0v = wi0[...], wh0[...], bi0[...] + bh0[...]
    wi1v, wh1v, b1v = wi1[...], wh1[...], bi1[...] + bh1[...]
    wi2v, wh2v, b2v = wi2[...], wh2[...], bi2[...] + bh2[...]
    l1wv, l2wv = l1w[...], l2w[...]
    zero = jnp.zeros((1, D), _f32)

    def lstm(g, cp):
        i_g = jax.nn.sigmoid(g[:, 0:D])
        f_g = jax.nn.sigmoid(g[:, D:2 * D])
        g_g = jnp.tanh(g[:, 2 * D:3 * D])
        o_g = jax.nn.sigmoid(g[:, 3 * D:4 * D])
        cn = f_g * cp + i_g * g_g
        return o_g * jnp.tanh(cn), cn

    def it(t, carry):
        q, r, h0, c0, h1, c1, h2, c2 = carry
        g = (jnp.dot(q, wi0v[0:D], preferred_element_type=_f32)
             + jnp.dot(r, wi0v[D:2 * D], preferred_element_type=_f32)
             + jnp.dot(h0, wh0v, preferred_element_type=_f32) + b0v)
        h0, c0 = lstm(g, c0)
        g = (jnp.dot(h0, wi1v, preferred_element_type=_f32)
             + jnp.dot(h1, wh1v, preferred_element_type=_f32) + b1v)
        h1, c1 = lstm(g, c1)
        g = (jnp.dot(h1, wi2v, preferred_element_type=_f32)
             + jnp.dot(h2, wh2v, preferred_element_type=_f32) + b2v)
        h2, c2 = lstm(g, c2)
        q = h2
        e = jnp.sum(x * q, axis=1, keepdims=True)
        a = jnp.exp(e - jnp.max(e))
        r = jnp.sum(a * x, axis=0, keepdims=True) / jnp.sum(a)
        return (q, r, h0, c0, h1, c1, h2, c2)

    q, r = lax.fori_loop(0, T_S2S, it, (zero,) * 8)[:2]
    y = jax.nn.relu(jnp.dot(q, l1wv[0:D], preferred_element_type=_f32)
                    + jnp.dot(r, l1wv[D:2 * D], preferred_element_type=_f32)
                    + l1b[...])
    y = jnp.dot(y, l2wv, preferred_element_type=_f32) + l2b[...]
    out_ref[...] = jnp.broadcast_to(y, (8, D))


_s2s_call = pl.pallas_call(
    _s2s_body,
    out_shape=jax.ShapeDtypeStruct((8, D), _f32),
)


def kernel(pos_undirected, pos_directed, params, nfreq, seed, efreq, edge_index):
    p = params
    src = edge_index[0].astype(jnp.int32)
    dst = edge_index[1].astype(jnp.int32)
    efq = efreq.astype(jnp.int32)

    ef_feat = jnp.concatenate(
        [p['edge_freq_emb'], (jnp.arange(NF, dtype=_f32) / MAX_EF)[:, None]], axis=1)
    ewtab, eidx = _const_call(
        ef_feat, p['edge_W1'], p['edge_b1'][None], p['edge_W2'], p['edge_b2'][None],
        src.reshape(EROWS, CHUNK), efq.reshape(EROWS, CHUNK))
    wbig = ewtab.reshape(NF, D, D).transpose(1, 0, 2).reshape(D, NF * D)
    dstp = dst.reshape(EROWS, CHUNK)

    feats = jnp.concatenate(
        [pos_undirected.astype(_f32), pos_directed.astype(_f32),
         seed.astype(_f32)[:, None], nfreq.astype(_f32)[:, None]], axis=1)
    w0 = p['lin0_W']
    wcat = jnp.concatenate(
        [w0[0:2 * P], w0[2 * P + D:2 * P + D + 1],
         w0[2 * P + D + 1:2 * P + D + 2] * (1.0 / MAX_NF)], axis=0)
    embw = jnp.dot(p['node_freq_emb'], w0[2 * P:2 * P + D],
                   preferred_element_type=_f32)
    h, outp = _init_call(feats, embw, wcat, p['lin0_b'][None], wbig)

    for _ in range(T_MP):
        aggs = _mp_call(outp.reshape(N * NF, D), eidx, dstp)
        h, outp = _gru_call(
            aggs, aggs, h, p['conv_bias'][None],
            p['gru_Wih'], p['gru_bih'][None], p['gru_Whh'], p['gru_bhh'][None],
            wbig)

    y8 = _s2s_call(
        h,
        p['lstm0_Wih'], p['lstm0_Whh'], p['lstm0_bih'][None], p['lstm0_bhh'][None],
        p['lstm1_Wih'], p['lstm1_Whh'], p['lstm1_bih'][None], p['lstm1_bhh'][None],
        p['lstm2_Wih'], p['lstm2_Whh'], p['lstm2_bih'][None], p['lstm2_bhh'][None],
        p['lin1_W'], p['lin1_b'][None], p['lin2_W'], p['lin2_b'][None])
    return y8[0:1]


# speedup vs baseline: 1.3382x; 1.0244x over previous
import jax
import jax.numpy as jnp
from jax import lax
from jax.experimental import pallas as pl
from jax.experimental.pallas import tpu as pltpu
from jax.experimental.pallas import tpu_sc as plsc

N = 10000
E = 160000
P = 16
D = 32
MAX_NF = 8
MAX_EF = 8
NF = MAX_EF + 1
T_MP = 3
T_S2S = 6

NC, NS = 2, 16
CHUNK = 125
EROWS = E // CHUNK
RPW = EROWS // (NC * NS)
NROWS = N
RPT = NROWS // NS
NBLK = 2000

_f32 = jnp.float32


def _const_body(ef, w1, b1, w2, b2, src, efq, ewtab, eidx):
    v = jax.nn.relu(jnp.dot(ef[...], w1[...], preferred_element_type=_f32) + b1[...])
    ewtab[...] = jnp.dot(v, w2[...], preferred_element_type=_f32) + b2[...]
    eidx[...] = src[...] * NF + jnp.clip(efq[...], 0, MAX_EF)


_const_call = pl.pallas_call(
    _const_body,
    out_shape=[jax.ShapeDtypeStruct((NF, D * D), _f32),
               jax.ShapeDtypeStruct((EROWS, CHUNK), jnp.int32)],
)


def _whole(shape):
    return pl.BlockSpec(shape, lambda i: (0,) * len(shape))


def _project(x, wbig, outp):
    outp[...] = jnp.dot(x, wbig, preferred_element_type=_f32)


FDIM = 2 * P + 2


def _init_body(feats, embw, wcat, b0, wbig, out0, outp):
    fv = feats[...]
    nfi = jnp.clip(fv[:, FDIM - 1:FDIM].astype(jnp.int32), 0, MAX_NF)
    oh = (lax.broadcasted_iota(jnp.int32, (NBLK, NF), 1) == nfi).astype(_f32)
    x = (jnp.dot(fv, wcat[...], preferred_element_type=_f32)
         + jnp.dot(oh, embw[...], preferred_element_type=_f32)
         + b0[...])
    o = jax.nn.relu(x)
    out0[...] = o
    _project(o, wbig[...], outp)


_init_call = pl.pallas_call(
    _init_body,
    grid=(N // NBLK,),
    in_specs=[pl.BlockSpec((NBLK, FDIM), lambda i: (i, 0)),
              _whole((NF, D)),
              _whole((FDIM, D)), _whole((1, D)),
              _whole((D, NF * D))],
    out_specs=[pl.BlockSpec((NBLK, D), lambda i: (i, 0)),
               pl.BlockSpec((NBLK, NF * D), lambda i: (i, 0))],
    out_shape=[jax.ShapeDtypeStruct((N, D), _f32),
               jax.ShapeDtypeStruct((N, NF * D), _f32)],
)


NBUF = 8


def _mp_body(outp_hbm, eidx_hbm, dst_hbm, aggs_hbm,
             zbuf, eidx_v, dst_v, rows_v, agg_sh, isem, gsem, ssem):
    cid = lax.axis_index("c")
    sid = lax.axis_index("s")

    base = cid * (EROWS // NC) + sid * RPW
    idx_cp = pltpu.async_copy(eidx_hbm.at[pl.ds(base, RPW)], eidx_v, isem)
    dst_cp = pltpu.async_copy(dst_hbm.at[pl.ds(base, RPW)], dst_v, isem)

    def zr(i, c):
        zbuf[i, pl.ds(0, 16)] = jnp.zeros((16,), _f32)
        zbuf[i, pl.ds(16, 16)] = jnp.zeros((16,), _f32)
        return c

    lax.fori_loop(0, RPT, zr, 0)
    idx_cp.wait()
    dst_cp.wait()
    for k in range(NBUF):
        pltpu.async_copy(outp_hbm.at[eidx_v.at[k]], rows_v.at[k], gsem.at[k])
    pltpu.sync_copy(zbuf, agg_sh.at[pl.ds(sid * RPT, RPT)])
    plsc.subcore_barrier()

    def round_(r, c):
        j0 = r * NBUF
        for k in range(NBUF):
            j = j0 + k
            pltpu.make_async_copy(
                outp_hbm.at[eidx_v.at[j]], rows_v.at[k], gsem.at[k]).wait()
            pltpu.async_copy(rows_v.at[k], agg_sh.at[dst_v.at[j]], ssem.at[k],
                             add=True)
        for k in range(NBUF):
            j = j0 + k

            @pl.when(j + NBUF < RPW)
            def _():
                pltpu.make_async_copy(
                    rows_v.at[k], agg_sh.at[dst_v.at[j]], ssem.at[k]).wait()
                pltpu.async_copy(outp_hbm.at[eidx_v.at[j + NBUF]],
                                 rows_v.at[k], gsem.at[k])
        return c

    lax.fori_loop(0, RPW // NBUF, round_, 0)
    for k in range(NBUF):
        j = RPW - NBUF + k
        pltpu.make_async_copy(
            rows_v.at[k], agg_sh.at[dst_v.at[j]], ssem.at[k]).wait()
    plsc.subcore_barrier()
    pltpu.sync_copy(agg_sh.at[pl.ds(sid * RPT, RPT)], zbuf)
    pltpu.sync_copy(zbuf, aggs_hbm.at[pl.ds(cid * NROWS + sid * RPT, RPT)])


_mp_call = pl.kernel(
    _mp_body,
    out_type=jax.ShapeDtypeStruct((NC * NROWS, D), _f32),
    mesh=plsc.VectorSubcoreMesh(core_axis_name="c", subcore_axis_name="s",
                                num_cores=NC, num_subcores=NS),
    scratch_types=[pltpu.VMEM((RPT, D), _f32),
                   pltpu.VMEM((RPW, CHUNK), jnp.int32),
                   pltpu.VMEM((RPW, CHUNK), jnp.int32),
                   pltpu.VMEM((NBUF, CHUNK, D), _f32),
                   pltpu.VMEM_SHARED((NROWS, D), _f32),
                   pltpu.SemaphoreType.DMA,
                   pltpu.SemaphoreType.DMA((NBUF,)),
                   pltpu.SemaphoreType.DMA((NBUF,))],
    compiler_params=pltpu.CompilerParams(use_tc_tiling_on_sc=False),
)


def _gru_body(a0, a1, h, cb, wih, bih, whh, bhh, wbig, hout, outp):
    hv = h[...]
    m = jax.nn.relu(a0[...] + a1[...] + cb[...])
    gi = jnp.dot(m, wih[...], preferred_element_type=_f32) + bih[...]
    gh = jnp.dot(hv, whh[...], preferred_element_type=_f32) + bhh[...]
    r = jax.nn.sigmoid(gi[:, 0:D] + gh[:, 0:D])
    z = jax.nn.sigmoid(gi[:, D:2 * D] + gh[:, D:2 * D])
    n_ = jnp.tanh(gi[:, 2 * D:3 * D] + r * gh[:, 2 * D:3 * D])
    hnew = (1.0 - z) * n_ + z * hv
    hout[...] = hnew
    _project(hnew, wbig[...], outp)


_gru_call = pl.pallas_call(
    _gru_body,
    grid=(N // NBLK,),
    in_specs=[pl.BlockSpec((NBLK, D), lambda i: (i, 0)),
              pl.BlockSpec((NBLK, D), lambda i: (i + NROWS // NBLK, 0)),
              pl.BlockSpec((NBLK, D), lambda i: (i, 0)),
              _whole((1, D)),
              _whole((D, 3 * D)), _whole((1, 3 * D)),
              _whole((D, 3 * D)), _whole((1, 3 * D)),
              _whole((D, NF * D))],
    out_specs=[pl.BlockSpec((NBLK, D), lambda i: (i, 0)),
               pl.BlockSpec((NBLK, NF * D), lambda i: (i, 0))],
    out_shape=[jax.ShapeDtypeStruct((N, D), _f32),
               jax.ShapeDtypeStruct((N, NF * D), _f32)],
)


def _s2s_body(h_ref, wi0, wh0, bi0, bh0, wi1, wh1, bi1, bh1,
              wi2, wh2, bi2, bh2, l1w, l1b, l2w, l2b, out_ref):
    x = h_ref[...]
    wi0v, wh0v, b0v = wi0[...], wh0[...], bi0[...] + bh0[...]
    wi1v, wh1v, b1v = wi1[...], wh1[...], bi1[...] + bh1[...]
    wi2v, wh2v, b2v = wi2[...], wh2[...], bi2[...] + bh2[...]
    l1wv, l2wv = l1w[...], l2w[...]
    zero = jnp.zeros((1, D), _f32)

    def lstm(g, cp):
        i_g = jax.nn.sigmoid(g[:, 0:D])
        f_g = jax.nn.sigmoid(g[:, D:2 * D])
        g_g = jnp.tanh(g[:, 2 * D:3 * D])
        o_g = jax.nn.sigmoid(g[:, 3 * D:4 * D])
        cn = f_g * cp + i_g * g_g
        return o_g * jnp.tanh(cn), cn

    def it(t, carry):
        q, r, h0, c0, h1, c1, h2, c2 = carry
        g = (jnp.dot(q, wi0v[0:D], preferred_element_type=_f32)
             + jnp.dot(r, wi0v[D:2 * D], preferred_element_type=_f32)
             + jnp.dot(h0, wh0v, preferred_element_type=_f32) + b0v)
        h0, c0 = lstm(g, c0)
        g = (jnp.dot(h0, wi1v, preferred_element_type=_f32)
             + jnp.dot(h1, wh1v, preferred_element_type=_f32) + b1v)
        h1, c1 = lstm(g, c1)
        g = (jnp.dot(h1, wi2v, preferred_element_type=_f32)
             + jnp.dot(h2, wh2v, preferred_element_type=_f32) + b2v)
        h2, c2 = lstm(g, c2)
        q = h2
        e = jnp.sum(x * q, axis=1, keepdims=True)
        a = jnp.exp(e - jnp.max(e))
        r = jnp.sum(a * x, axis=0, keepdims=True) / jnp.sum(a)
        return (q, r, h0, c0, h1, c1, h2, c2)

    q, r = lax.fori_loop(0, T_S2S, it, (zero,) * 8)[:2]
    y = jax.nn.relu(jnp.dot(q, l1wv[0:D], preferred_element_type=_f32)
                    + jnp.dot(r, l1wv[D:2 * D], preferred_element_type=_f32)
                    + l1b[...])
    y = jnp.dot(y, l2wv, preferred_element_type=_f32) + l2b[...]
    out_ref[...] = jnp.broadcast_to(y, (8, D))


_s2s_call = pl.pallas_call(
    _s2s_body,
    out_shape=jax.ShapeDtypeStruct((8, D), _f32),
)


def kernel(pos_undirected, pos_directed, params, nfreq, seed, efreq, edge_index):
    p = params
    src = edge_index[0].astype(jnp.int32)
    dst = edge_index[1].astype(jnp.int32)
    efq = efreq.astype(jnp.int32)

    ef_feat = jnp.concatenate(
        [p['edge_freq_emb'], (jnp.arange(NF, dtype=_f32) / MAX_EF)[:, None]], axis=1)
    ewtab, eidx = _const_call(
        ef_feat, p['edge_W1'], p['edge_b1'][None], p['edge_W2'], p['edge_b2'][None],
        src.reshape(EROWS, CHUNK), efq.reshape(EROWS, CHUNK))
    wbig = ewtab.reshape(NF, D, D).transpose(1, 0, 2).reshape(D, NF * D)
    dstp = dst.reshape(EROWS, CHUNK)

    feats = jnp.concatenate(
        [pos_undirected.astype(_f32), pos_directed.astype(_f32),
         seed.astype(_f32)[:, None], nfreq.astype(_f32)[:, None]], axis=1)
    w0 = p['lin0_W']
    wcat = jnp.concatenate(
        [w0[0:2 * P], w0[2 * P + D:2 * P + D + 1],
         w0[2 * P + D + 1:2 * P + D + 2] * (1.0 / MAX_NF)], axis=0)
    embw = jnp.dot(p['node_freq_emb'], w0[2 * P:2 * P + D],
                   preferred_element_type=_f32)
    h, outp = _init_call(feats, embw, wcat, p['lin0_b'][None], wbig)

    for _ in range(T_MP):
        aggs = _mp_call(outp.reshape(N * NF, D), eidx, dstp)
        h, outp = _gru_call(
            aggs, aggs, h, p['conv_bias'][None],
            p['gru_Wih'], p['gru_bih'][None], p['gru_Whh'], p['gru_bhh'][None],
            wbig)

    y8 = _s2s_call(
        h,
        p['lstm0_Wih'], p['lstm0_Whh'], p['lstm0_bih'][None], p['lstm0_bhh'][None],
        p['lstm1_Wih'], p['lstm1_Whh'], p['lstm1_bih'][None], p['lstm1_bhh'][None],
        p['lstm2_Wih'], p['lstm2_Whh'], p['lstm2_bih'][None], p['lstm2_bhh'][None],
        p['lin1_W'], p['lin1_b'][None], p['lin2_W'], p['lin2_b'][None])
    return y8[0:1]


# speedup vs baseline: 1.3404x; 1.0017x over previous
import jax
import jax.numpy as jnp
from jax import lax
from jax.experimental import pallas as pl
from jax.experimental.pallas import tpu as pltpu
from jax.experimental.pallas import tpu_sc as plsc

N = 10000
E = 160000
P = 16
D = 32
MAX_NF = 8
MAX_EF = 8
NF = MAX_EF + 1
T_MP = 3
T_S2S = 6

NC, NS = 2, 16
CHUNK = 125
EROWS = E // CHUNK
RPW = EROWS // (NC * NS)
NROWS = N
RPT = NROWS // NS
NBLK = 2000

_f32 = jnp.float32


def _const_body(ef, w1, b1, w2, b2, src, efq, ewtab, eidx):
    v = jax.nn.relu(jnp.dot(ef[...], w1[...], preferred_element_type=_f32) + b1[...])
    ewtab[...] = jnp.dot(v, w2[...], preferred_element_type=_f32) + b2[...]
    eidx[...] = src[...] * NF + jnp.clip(efq[...], 0, MAX_EF)


_const_call = pl.pallas_call(
    _const_body,
    out_shape=[jax.ShapeDtypeStruct((NF, D * D), _f32),
               jax.ShapeDtypeStruct((EROWS, CHUNK), jnp.int32)],
)


def _whole(shape):
    return pl.BlockSpec(shape, lambda i: (0,) * len(shape))


def _project(x, wbig, outp):
    outp[...] = jnp.dot(x, wbig, preferred_element_type=_f32)


FDIM = 2 * P + 2


def _init_body(feats, embw, wcat, b0, wbig, out0, outp):
    fv = feats[...]
    nfi = jnp.clip(fv[:, FDIM - 1:FDIM].astype(jnp.int32), 0, MAX_NF)
    oh = (lax.broadcasted_iota(jnp.int32, (NBLK, NF), 1) == nfi).astype(_f32)
    x = (jnp.dot(fv, wcat[...], preferred_element_type=_f32)
         + jnp.dot(oh, embw[...], preferred_element_type=_f32)
         + b0[...])
    o = jax.nn.relu(x)
    out0[...] = o
    _project(o, wbig[...], outp)


_init_call = pl.pallas_call(
    _init_body,
    grid=(N // NBLK,),
    in_specs=[pl.BlockSpec((NBLK, FDIM), lambda i: (i, 0)),
              _whole((NF, D)),
              _whole((FDIM, D)), _whole((1, D)),
              _whole((D, NF * D))],
    out_specs=[pl.BlockSpec((NBLK, D), lambda i: (i, 0)),
               pl.BlockSpec((NBLK, NF * D), lambda i: (i, 0))],
    out_shape=[jax.ShapeDtypeStruct((N, D), _f32),
               jax.ShapeDtypeStruct((N, NF * D), _f32)],
)


NBUF = 10


def _mp_body(outp_hbm, eidx_hbm, dst_hbm, aggs_hbm,
             zbuf, eidx_v, dst_v, rows_v, agg_sh, isem, gsem, ssem):
    cid = lax.axis_index("c")
    sid = lax.axis_index("s")

    base = cid * (EROWS // NC) + sid * RPW
    idx_cp = pltpu.async_copy(eidx_hbm.at[pl.ds(base, RPW)], eidx_v, isem)
    dst_cp = pltpu.async_copy(dst_hbm.at[pl.ds(base, RPW)], dst_v, isem)

    def zr(i, c):
        zbuf[i, pl.ds(0, 16)] = jnp.zeros((16,), _f32)
        zbuf[i, pl.ds(16, 16)] = jnp.zeros((16,), _f32)
        return c

    lax.fori_loop(0, RPT, zr, 0)
    idx_cp.wait()
    dst_cp.wait()
    for k in range(NBUF):
        pltpu.async_copy(outp_hbm.at[eidx_v.at[k]], rows_v.at[k], gsem.at[k])
    pltpu.sync_copy(zbuf, agg_sh.at[pl.ds(sid * RPT, RPT)])
    plsc.subcore_barrier()

    def round_(r, c):
        j0 = r * NBUF
        for k in range(NBUF):
            j = j0 + k
            pltpu.make_async_copy(
                outp_hbm.at[eidx_v.at[j]], rows_v.at[k], gsem.at[k]).wait()
            pltpu.async_copy(rows_v.at[k], agg_sh.at[dst_v.at[j]], ssem.at[k],
                             add=True)
        for k in range(NBUF):
            j = j0 + k

            @pl.when(j + NBUF < RPW)
            def _():
                pltpu.make_async_copy(
                    rows_v.at[k], agg_sh.at[dst_v.at[j]], ssem.at[k]).wait()
                pltpu.async_copy(outp_hbm.at[eidx_v.at[j + NBUF]],
                                 rows_v.at[k], gsem.at[k])
        return c

    lax.fori_loop(0, RPW // NBUF, round_, 0)
    for k in range(NBUF):
        j = RPW - NBUF + k
        pltpu.make_async_copy(
            rows_v.at[k], agg_sh.at[dst_v.at[j]], ssem.at[k]).wait()
    plsc.subcore_barrier()
    pltpu.sync_copy(agg_sh.at[pl.ds(sid * RPT, RPT)], zbuf)
    pltpu.sync_copy(zbuf, aggs_hbm.at[pl.ds(cid * NROWS + sid * RPT, RPT)])


_mp_call = pl.kernel(
    _mp_body,
    out_type=jax.ShapeDtypeStruct((NC * NROWS, D), _f32),
    mesh=plsc.VectorSubcoreMesh(core_axis_name="c", subcore_axis_name="s",
                                num_cores=NC, num_subcores=NS),
    scratch_types=[pltpu.VMEM((RPT, D), _f32),
                   pltpu.VMEM((RPW, CHUNK), jnp.int32),
                   pltpu.VMEM((RPW, CHUNK), jnp.int32),
                   pltpu.VMEM((NBUF, CHUNK, D), _f32),
                   pltpu.VMEM_SHARED((NROWS, D), _f32),
                   pltpu.SemaphoreType.DMA,
                   pltpu.SemaphoreType.DMA((NBUF,)),
                   pltpu.SemaphoreType.DMA((NBUF,))],
    compiler_params=pltpu.CompilerParams(use_tc_tiling_on_sc=False),
)


def _gru_body(a0, a1, h, cb, wih, bih, whh, bhh, wbig, hout, outp):
    hv = h[...]
    m = jax.nn.relu(a0[...] + a1[...] + cb[...])
    gi = jnp.dot(m, wih[...], preferred_element_type=_f32) + bih[...]
    gh = jnp.dot(hv, whh[...], preferred_element_type=_f32) + bhh[...]
    r = jax.nn.sigmoid(gi[:, 0:D] + gh[:, 0:D])
    z = jax.nn.sigmoid(gi[:, D:2 * D] + gh[:, D:2 * D])
    n_ = jnp.tanh(gi[:, 2 * D:3 * D] + r * gh[:, 2 * D:3 * D])
    hnew = (1.0 - z) * n_ + z * hv
    hout[...] = hnew
    _project(hnew, wbig[...], outp)


_gru_call = pl.pallas_call(
    _gru_body,
    grid=(N // NBLK,),
    in_specs=[pl.BlockSpec((NBLK, D), lambda i: (i, 0)),
              pl.BlockSpec((NBLK, D), lambda i: (i + NROWS // NBLK, 0)),
              pl.BlockSpec((NBLK, D), lambda i: (i, 0)),
              _whole((1, D)),
              _whole((D, 3 * D)), _whole((1, 3 * D)),
              _whole((D, 3 * D)), _whole((1, 3 * D)),
              _whole((D, NF * D))],
    out_specs=[pl.BlockSpec((NBLK, D), lambda i: (i, 0)),
               pl.BlockSpec((NBLK, NF * D), lambda i: (i, 0))],
    out_shape=[jax.ShapeDtypeStruct((N, D), _f32),
               jax.ShapeDtypeStruct((N, NF * D), _f32)],
)


def _s2s_body(h_ref, wi0, wh0, bi0, bh0, wi1, wh1, bi1, bh1,
              wi2, wh2, bi2, bh2, l1w, l1b, l2w, l2b, out_ref):
    x = h_ref[...]
    wi0v, wh0v, b0v = wi0[...], wh0[...], bi0[...] + bh0[...]
    wi1v, wh1v, b1v = wi1[...], wh1[...], bi1[...] + bh1[...]
    wi2v, wh2v, b2v = wi2[...], wh2[...], bi2[...] + bh2[...]
    l1wv, l2wv = l1w[...], l2w[...]
    zero = jnp.zeros((1, D), _f32)

    def lstm(g, cp):
        i_g = jax.nn.sigmoid(g[:, 0:D])
        f_g = jax.nn.sigmoid(g[:, D:2 * D])
        g_g = jnp.tanh(g[:, 2 * D:3 * D])
        o_g = jax.nn.sigmoid(g[:, 3 * D:4 * D])
        cn = f_g * cp + i_g * g_g
        return o_g * jnp.tanh(cn), cn

    def it(t, carry):
        q, r, h0, c0, h1, c1, h2, c2 = carry
        g = (jnp.dot(q, wi0v[0:D], preferred_element_type=_f32)
             + jnp.dot(r, wi0v[D:2 * D], preferred_element_type=_f32)
             + jnp.dot(h0, wh0v, preferred_element_type=_f32) + b0v)
        h0, c0 = lstm(g, c0)
        g = (jnp.dot(h0, wi1v, preferred_element_type=_f32)
             + jnp.dot(h1, wh1v, preferred_element_type=_f32) + b1v)
        h1, c1 = lstm(g, c1)
        g = (jnp.dot(h1, wi2v, preferred_element_type=_f32)
             + jnp.dot(h2, wh2v, preferred_element_type=_f32) + b2v)
        h2, c2 = lstm(g, c2)
        q = h2
        e = jnp.sum(x * q, axis=1, keepdims=True)
        a = jnp.exp(e - jnp.max(e))
        r = jnp.sum(a * x, axis=0, keepdims=True) / jnp.sum(a)
        return (q, r, h0, c0, h1, c1, h2, c2)

    q, r = lax.fori_loop(0, T_S2S, it, (zero,) * 8)[:2]
    y = jax.nn.relu(jnp.dot(q, l1wv[0:D], preferred_element_type=_f32)
                    + jnp.dot(r, l1wv[D:2 * D], preferred_element_type=_f32)
                    + l1b[...])
    y = jnp.dot(y, l2wv, preferred_element_type=_f32) + l2b[...]
    out_ref[...] = jnp.broadcast_to(y, (8, D))


_s2s_call = pl.pallas_call(
    _s2s_body,
    out_shape=jax.ShapeDtypeStruct((8, D), _f32),
)


def kernel(pos_undirected, pos_directed, params, nfreq, seed, efreq, edge_index):
    p = params
    src = edge_index[0].astype(jnp.int32)
    dst = edge_index[1].astype(jnp.int32)
    efq = efreq.astype(jnp.int32)

    ef_feat = jnp.concatenate(
        [p['edge_freq_emb'], (jnp.arange(NF, dtype=_f32) / MAX_EF)[:, None]], axis=1)
    ewtab, eidx = _const_call(
        ef_feat, p['edge_W1'], p['edge_b1'][None], p['edge_W2'], p['edge_b2'][None],
        src.reshape(EROWS, CHUNK), efq.reshape(EROWS, CHUNK))
    wbig = ewtab.reshape(NF, D, D).transpose(1, 0, 2).reshape(D, NF * D)
    dstp = dst.reshape(EROWS, CHUNK)

    feats = jnp.concatenate(
        [pos_undirected.astype(_f32), pos_directed.astype(_f32),
         seed.astype(_f32)[:, None], nfreq.astype(_f32)[:, None]], axis=1)
    w0 = p['lin0_W']
    wcat = jnp.concatenate(
        [w0[0:2 * P], w0[2 * P + D:2 * P + D + 1],
         w0[2 * P + D + 1:2 * P + D + 2] * (1.0 / MAX_NF)], axis=0)
    embw = jnp.dot(p['node_freq_emb'], w0[2 * P:2 * P + D],
                   preferred_element_type=_f32)
    h, outp = _init_call(feats, embw, wcat, p['lin0_b'][None], wbig)

    for _ in range(T_MP):
        aggs = _mp_call(outp.reshape(N * NF, D), eidx, dstp)
        h, outp = _gru_call(
            aggs, aggs, h, p['conv_bias'][None],
            p['gru_Wih'], p['gru_bih'][None], p['gru_Whh'], p['gru_bhh'][None],
            wbig)

    y8 = _s2s_call(
        h,
        p['lstm0_Wih'], p['lstm0_Whh'], p['lstm0_bih'][None], p['lstm0_bhh'][None],
        p['lstm1_Wih'], p['lstm1_Whh'], p['lstm1_bih'][None], p['lstm1_bhh'][None],
        p['lstm2_Wih'], p['lstm2_Whh'], p['lstm2_bih'][None], p['lstm2_bhh'][None],
        p['lin1_W'], p['lin1_b'][None], p['lin2_W'], p['lin2_b'][None])
    return y8[0:1]
